# baseline (speedup 1.0000x reference)
import functools

import jax
import jax.numpy as jnp
from jax import lax
from jax.experimental import pallas as pl
from jax.experimental.pallas import tpu as pltpu
from jax.experimental.pallas import tpu_sc as plsc

_NC = 2
_NS = 16
_NW = _NC * _NS
_L = 16
_G = 16


def _mf_kernel(B, F, u_emb3, i_emb3, u_bias1, i_bias1, u_idx, i_idx):
    b_per_w = B // _NW
    ng = b_per_w // _G
    nrow = b_per_w // 128
    group_bytes = 2 * _G * (2 * F) * 4
    drain_words = group_bytes // 4
    mesh = plsc.VectorSubcoreMesh(core_axis_name="c", subcore_axis_name="s")
    cp = pltpu.CompilerParams(needs_layout_passes=False)

    @functools.partial(
        pl.kernel,
        mesh=mesh,
        compiler_params=cp,
        out_type=jax.ShapeDtypeStruct((B,), jnp.float32),
        scratch_types=[
            pltpu.VMEM((b_per_w // 128, 128), jnp.int32),
            pltpu.VMEM((b_per_w // 128, 128), jnp.int32),
            pltpu.VMEM((2, _G, 128), jnp.float32),
            pltpu.VMEM((2, _G, 128), jnp.float32),
            pltpu.VMEM((b_per_w,), jnp.float32),
            pltpu.VMEM((b_per_w,), jnp.float32),
            pltpu.VMEM((drain_words,), jnp.int32),
            pltpu.VMEM((b_per_w,), jnp.float32),
            pltpu.SemaphoreType.DMA,
            pltpu.SemaphoreType.DMA,
            pltpu.SemaphoreType.DMA,
        ],
    )
    def k(u3, i3, ub_hbm, ib_hbm, u_idx_hbm, i_idx_hbm,
          out_hbm, uidx_v, iidx_v, u_ring, i_ring, ub_v, ib_v, drain_v,
          out_v, sem0, sem1, bsem):
        wid = lax.axis_index("s") * _NC + lax.axis_index("c")
        base = wid * b_per_w

        for kk in range(nrow):
            pltpu.sync_copy(u_idx_hbm.at[pl.ds(base + kk * 128, 128)],
                            uidx_v.at[kk])
            pltpu.sync_copy(i_idx_hbm.at[pl.ds(base + kk * 128, 128)],
                            iidx_v.at[kk])

        for kk in range(nrow):
            pltpu.async_copy(ub_hbm.at[uidx_v.at[kk]],
                             ub_v.at[pl.ds(kk * 128, 128)], bsem)
            pltpu.async_copy(ib_hbm.at[iidx_v.at[kk]],
                             ib_v.at[pl.ds(kk * 128, 128)], bsem)

        def enqueue(g, buf, sem):
            rb = g * _G
            uidx16 = uidx_v[rb // 128, pl.ds(rb % 128, _G)]
            iidx16 = iidx_v[rb // 128, pl.ds(rb % 128, _G)]
            up16 = lax.shift_right_logical(uidx16, 1)
            ip16 = lax.shift_right_logical(iidx16, 1)
            for j in range(_G):
                pltpu.async_copy(u3.at[up16[j]], u_ring.at[buf, j], sem)
                pltpu.async_copy(i3.at[ip16[j]], i_ring.at[buf, j], sem)

        def drain(sem):
            pltpu.make_async_copy(
                u_idx_hbm.at[pl.ds(0, drain_words)], drain_v, sem).wait()

        lane = lax.iota(jnp.int32, _L)

        def compute(g, buf):
            rb = g * _G
            uidx16 = uidx_v[rb // 128, pl.ds(rb % 128, _G)]
            iidx16 = iidx_v[rb // 128, pl.ds(rb % 128, _G)]
            uh16 = lax.bitwise_and(uidx16, 1) * F
            ih16 = lax.bitwise_and(iidx16, 1) * F
            out16 = ub_v[pl.ds(rb, _G)] + ib_v[pl.ds(rb, _G)]
            for j in range(_G):
                uh, ih = uh16[j], ih16[j]
                acc = (u_ring[buf, j, pl.ds(uh, _L)]
                       * i_ring[buf, j, pl.ds(ih, _L)])
                for fb in range(1, F // _L):
                    acc = acc + (u_ring[buf, j, pl.ds(uh + fb * _L, _L)]
                                 * i_ring[buf, j, pl.ds(ih + fb * _L, _L)])
                out16 = out16 + jnp.where(lane == j, jnp.sum(acc), 0.0)
            out_v[pl.ds(rb, _G)] = out16

        enqueue(0, 0, sem0)
        for kk in range(2 * nrow):
            pltpu.make_async_copy(
                u_idx_hbm.at[pl.ds(0, 128)],
                drain_v.at[pl.ds(0, 128)], bsem).wait()

        @pl.loop(0, ng // 2)
        def _(kk):
            g0 = kk * 2
            enqueue(g0 + 1, 1, sem1)
            drain(sem0)
            compute(g0, 0)

            @pl.when(g0 + 2 < ng)
            def _():
                enqueue(g0 + 2, 0, sem0)

            drain(sem1)
            compute(g0 + 1, 1)

        pltpu.sync_copy(out_v, out_hbm.at[pl.ds(base, b_per_w)])

    return k(u_emb3, i_emb3, u_bias1, i_bias1, u_idx, i_idx)


@jax.jit
def kernel(u_emb, i_emb, u_bias, i_bias, u_idx, i_idx):
    B = u_idx.shape[0]
    F = u_emb.shape[1]
    u3 = u_emb.reshape(-1, 2 * F)
    i3 = i_emb.reshape(-1, 2 * F)
    ub1 = u_bias.reshape(-1)
    ib1 = i_bias.reshape(-1)
    return _mf_kernel(
        B, F, u3, i3, ub1, ib1,
        u_idx.astype(jnp.int32), i_idx.astype(jnp.int32),
    )


# speedup vs baseline: 2.0667x; 2.0667x over previous
import functools

import jax
import jax.numpy as jnp
from jax import lax
from jax.experimental import pallas as pl
from jax.experimental.pallas import tpu as pltpu
from jax.experimental.pallas import tpu_sc as plsc

_NC = 2
_NS = 16
_NW = _NC * _NS
_L = 16
_G = 16
_R = 4


def _mf_kernel(B, F, u3, i3, u_bias1, i_bias1, u_idx, i_idx):
    b_per_w = B // _NW
    ng = b_per_w // _G
    nrow = b_per_w // 128
    nf1 = F // 8
    mesh = plsc.VectorSubcoreMesh(core_axis_name="c", subcore_axis_name="s")
    cp = pltpu.CompilerParams(needs_layout_passes=False)

    @functools.partial(
        pl.kernel,
        mesh=mesh,
        compiler_params=cp,
        out_type=jax.ShapeDtypeStruct((B,), jnp.float32),
        scratch_types=[
            pltpu.VMEM((b_per_w // 128, 128), jnp.int32),
            pltpu.VMEM((b_per_w // 128, 128), jnp.int32),
            pltpu.VMEM((_R, nf1, 8, 128), jnp.float32),
            pltpu.VMEM((_R, nf1, 8, 128), jnp.float32),
            pltpu.VMEM((b_per_w,), jnp.float32),
            pltpu.VMEM((b_per_w,), jnp.float32),
            pltpu.VMEM((128,), jnp.int32),
            pltpu.VMEM((b_per_w,), jnp.float32),
            pltpu.SemaphoreType.DMA,
            pltpu.SemaphoreType.DMA,
            pltpu.SemaphoreType.DMA,
            pltpu.SemaphoreType.DMA,
            pltpu.SemaphoreType.DMA,
        ],
    )
    def k(u3r, i3r, ub_hbm, ib_hbm, u_idx_hbm, i_idx_hbm,
          out_hbm, uidx_v, iidx_v, u_ring, i_ring, ub_v, ib_v, drain_v,
          out_v, sem0, sem1, sem2, sem3, bsem):
        sems = [sem0, sem1, sem2, sem3]
        wid = lax.axis_index("s") * _NC + lax.axis_index("c")
        base = wid * b_per_w

        for kk in range(nrow):
            pltpu.sync_copy(u_idx_hbm.at[pl.ds(base + kk * 128, 128)],
                            uidx_v.at[kk])
            pltpu.sync_copy(i_idx_hbm.at[pl.ds(base + kk * 128, 128)],
                            iidx_v.at[kk])

        for kk in range(nrow):
            pltpu.async_copy(ub_hbm.at[uidx_v.at[kk]],
                             ub_v.at[pl.ds(kk * 128, 128)], bsem)
            pltpu.async_copy(ib_hbm.at[iidx_v.at[kk]],
                             ib_v.at[pl.ds(kk * 128, 128)], bsem)

        def idx_vecs(g):
            rb = g * _G
            uidx16 = uidx_v[rb // 128, pl.ds(rb % 128, _G)]
            iidx16 = iidx_v[rb // 128, pl.ds(rb % 128, _G)]
            return uidx16, iidx16

        def enqueue(uidx16, iidx16, j, slot):
            uc = pl.multiple_of(
                lax.shift_right_logical(uidx16[j], 7) * jnp.int32(128), 128)
            ic = pl.multiple_of(
                lax.shift_right_logical(iidx16[j], 7) * jnp.int32(128), 128)
            pltpu.async_copy(u3r.at[:, :, pl.ds(uc, 128)],
                             u_ring.at[slot], sems[slot])
            pltpu.async_copy(i3r.at[:, :, pl.ds(ic, 128)],
                             i_ring.at[slot], sems[slot])

        def wait_slot(slot):
            pltpu.make_async_copy(u3r.at[:, :, pl.ds(0, 128)],
                                  u_ring.at[slot], sems[slot]).wait()
            pltpu.make_async_copy(i3r.at[:, :, pl.ds(0, 128)],
                                  i_ring.at[slot], sems[slot]).wait()

        lane = lax.iota(jnp.int32, _L)
        f2v = lax.bitwise_and(lane, 7)
        f1v = lax.shift_right_logical(lane, 3)

        def compute(uidx16, iidx16, j, slot, out16):
            ucb = jnp.zeros((_L,), jnp.int32) + lax.bitwise_and(uidx16[j], 127)
            icb = jnp.zeros((_L,), jnp.int32) + lax.bitwise_and(iidx16[j], 127)
            acc = None
            for q in range(F // _L):
                f1q = f1v + (2 * q)
                uv = plsc.load_gather(u_ring.at[slot], [f1q, f2v, ucb])
                iv = plsc.load_gather(i_ring.at[slot], [f1q, f2v, icb])
                prod = uv * iv
                acc = prod if acc is None else acc + prod
            return out16 + jnp.where(lane == j, jnp.sum(acc), 0.0)

        for kk in range(2 * nrow):
            pltpu.make_async_copy(
                u_idx_hbm.at[pl.ds(0, 128)],
                drain_v.at[pl.ds(0, 128)], bsem).wait()

        u0, i0 = idx_vecs(0)
        enqueue(u0, i0, 0, 0)
        enqueue(u0, i0, 1, 1)

        @pl.loop(0, ng)
        def _(g):
            uidx16, iidx16 = idx_vecs(g)
            un, inn = idx_vecs(jnp.minimum(g + 1, ng - 1))
            rb = g * _G
            out16 = ub_v[pl.ds(rb, _G)] + ib_v[pl.ds(rb, _G)]
            for j in range(_G):
                slot = (j + 2) % _R
                if j < _G - 2:
                    enqueue(uidx16, iidx16, j + 2, slot)
                else:
                    @pl.when(g + 1 < ng)
                    def _():
                        enqueue(un, inn, j + 2 - _G, slot)
                wait_slot(j % _R)
                out16 = compute(uidx16, iidx16, j, j % _R, out16)
            out_v[pl.ds(rb, _G)] = out16

        pltpu.sync_copy(out_v, out_hbm.at[pl.ds(base, b_per_w)])

    return k(u3, i3, u_bias1, i_bias1, u_idx, i_idx)


@jax.jit
def kernel(u_emb, i_emb, u_bias, i_bias, u_idx, i_idx):
    B = u_idx.shape[0]
    F = u_emb.shape[1]
    N = u_emb.shape[0]
    u3 = u_emb.T.reshape(F // 8, 8, N)
    i3 = i_emb.T.reshape(F // 8, 8, N)
    ub1 = u_bias.reshape(-1)
    ib1 = i_bias.reshape(-1)
    return _mf_kernel(
        B, F, u3, i3, ub1, ib1,
        u_idx.astype(jnp.int32), i_idx.astype(jnp.int32),
    )


# speedup vs baseline: 2.2209x; 1.0746x over previous
import functools

import jax
import jax.numpy as jnp
from jax import lax
from jax.experimental import pallas as pl
from jax.experimental.pallas import tpu as pltpu
from jax.experimental.pallas import tpu_sc as plsc

_NC = 2
_NS = 16
_NW = _NC * _NS
_L = 16
_G = 16
_R = 4


def _mf_kernel(B, F, u3, i3, u_bias1, i_bias1, u_idx, i_idx):
    b_per_w = B // _NW
    ng = b_per_w // _G
    nrow = b_per_w // 128
    nf1 = F // 8
    mesh = plsc.VectorSubcoreMesh(core_axis_name="c", subcore_axis_name="s")
    cp = pltpu.CompilerParams(needs_layout_passes=False)

    @functools.partial(
        pl.kernel,
        mesh=mesh,
        compiler_params=cp,
        out_type=jax.ShapeDtypeStruct((B,), jnp.float32),
        scratch_types=[
            pltpu.VMEM((b_per_w // 128, 128), jnp.int32),
            pltpu.VMEM((b_per_w // 128, 128), jnp.int32),
            pltpu.VMEM((_R, nf1, 8, 128), jnp.float32),
            pltpu.VMEM((_R, nf1, 8, 128), jnp.float32),
            pltpu.VMEM((b_per_w,), jnp.float32),
            pltpu.VMEM((b_per_w,), jnp.float32),
            pltpu.VMEM((128,), jnp.int32),
            pltpu.VMEM((b_per_w,), jnp.float32),
            pltpu.SemaphoreType.DMA,
            pltpu.SemaphoreType.DMA,
            pltpu.SemaphoreType.DMA,
            pltpu.SemaphoreType.DMA,
            pltpu.SemaphoreType.DMA,
        ],
    )
    def k(u3r, i3r, ub_hbm, ib_hbm, u_idx_hbm, i_idx_hbm,
          out_hbm, uidx_v, iidx_v, u_ring, i_ring, ub_v, ib_v, drain_v,
          out_v, sem0, sem1, sem2, sem3, bsem):
        sems = [sem0, sem1, sem2, sem3]
        wid = lax.axis_index("s") * _NC + lax.axis_index("c")
        base = wid * b_per_w

        for kk in range(nrow):
            pltpu.sync_copy(u_idx_hbm.at[pl.ds(base + kk * 128, 128)],
                            uidx_v.at[kk])
            pltpu.sync_copy(i_idx_hbm.at[pl.ds(base + kk * 128, 128)],
                            iidx_v.at[kk])

        for kk in range(nrow):
            pltpu.async_copy(ub_hbm.at[uidx_v.at[kk]],
                             ub_v.at[pl.ds(kk * 128, 128)], bsem)
            pltpu.async_copy(ib_hbm.at[iidx_v.at[kk]],
                             ib_v.at[pl.ds(kk * 128, 128)], bsem)

        def idx_vecs(g):
            rb = g * _G
            uidx16 = uidx_v[rb // 128, pl.ds(rb % 128, _G)]
            iidx16 = iidx_v[rb // 128, pl.ds(rb % 128, _G)]
            return uidx16, iidx16

        def enqueue(uidx16, iidx16, j, slot):
            uc = pl.multiple_of(
                lax.shift_right_logical(uidx16[j], 7) * jnp.int32(128), 128)
            ic = pl.multiple_of(
                lax.shift_right_logical(iidx16[j], 7) * jnp.int32(128), 128)
            pltpu.async_copy(u3r.at[:, :, pl.ds(uc, 128)],
                             u_ring.at[slot], sems[slot])
            pltpu.async_copy(i3r.at[:, :, pl.ds(ic, 128)],
                             i_ring.at[slot], sems[slot])

        def wait_slot(slot):
            pltpu.make_async_copy(u3r.at[:, :, pl.ds(0, 128)],
                                  u_ring.at[slot], sems[slot]).wait()
            pltpu.make_async_copy(i3r.at[:, :, pl.ds(0, 128)],
                                  i_ring.at[slot], sems[slot]).wait()

        lane = lax.iota(jnp.int32, _L)
        f2v = lax.bitwise_and(lane, 7)
        f1v = lax.shift_right_logical(lane, 3)

        def compute(uidx16, iidx16, j, slot, out16):
            ucb = jnp.zeros((_L,), jnp.int32) + lax.bitwise_and(uidx16[j], 127)
            icb = jnp.zeros((_L,), jnp.int32) + lax.bitwise_and(iidx16[j], 127)
            acc = None
            for q in range(F // _L):
                f1q = f1v + (2 * q)
                uv = plsc.load_gather(u_ring.at[slot], [f1q, f2v, ucb])
                iv = plsc.load_gather(i_ring.at[slot], [f1q, f2v, icb])
                prod = uv * iv
                acc = prod if acc is None else acc + prod
            return out16 + jnp.where(lane == j, jnp.sum(acc), 0.0)

        for kk in range(2 * nrow):
            pltpu.make_async_copy(
                u_idx_hbm.at[pl.ds(0, 128)],
                drain_v.at[pl.ds(0, 128)], bsem).wait()

        u0, i0 = idx_vecs(0)
        enqueue(u0, i0, 0, 0)
        enqueue(u0, i0, 1, 1)
        enqueue(u0, i0, 2, 2)

        @pl.loop(0, ng)
        def _(g):
            uidx16, iidx16 = idx_vecs(g)
            un, inn = idx_vecs(jnp.minimum(g + 1, ng - 1))
            rb = g * _G
            out16 = ub_v[pl.ds(rb, _G)] + ib_v[pl.ds(rb, _G)]
            for j in range(_G):
                slot = (j + 3) % _R
                if j < _G - 3:
                    enqueue(uidx16, iidx16, j + 3, slot)
                else:
                    @pl.when(g + 1 < ng)
                    def _():
                        enqueue(un, inn, j + 3 - _G, slot)
                wait_slot(j % _R)
                out16 = compute(uidx16, iidx16, j, j % _R, out16)
            out_v[pl.ds(rb, _G)] = out16

        pltpu.sync_copy(out_v, out_hbm.at[pl.ds(base, b_per_w)])

    return k(u3, i3, u_bias1, i_bias1, u_idx, i_idx)


@jax.jit
def kernel(u_emb, i_emb, u_bias, i_bias, u_idx, i_idx):
    B = u_idx.shape[0]
    F = u_emb.shape[1]
    N = u_emb.shape[0]
    u3 = u_emb.T.reshape(F // 8, 8, N)
    i3 = i_emb.T.reshape(F // 8, 8, N)
    ub1 = u_bias.reshape(-1)
    ib1 = i_bias.reshape(-1)
    return _mf_kernel(
        B, F, u3, i3, ub1, ib1,
        u_idx.astype(jnp.int32), i_idx.astype(jnp.int32),
    )


# speedup vs baseline: 2.2305x; 1.0043x over previous
import functools

import jax
import jax.numpy as jnp
from jax import lax
from jax.experimental import pallas as pl
from jax.experimental.pallas import tpu as pltpu
from jax.experimental.pallas import tpu_sc as plsc

_NC = 2
_NS = 16
_NW = _NC * _NS
_L = 16
_G = 16
_R = 4


def _mf_kernel(B, F, u3, i3, u_bias1, i_bias1, u_idx, i_idx):
    b_per_w = B // _NW
    ng = b_per_w // _G
    nrow = b_per_w // 128
    nf1 = F // 8
    mesh = plsc.VectorSubcoreMesh(core_axis_name="c", subcore_axis_name="s")
    cp = pltpu.CompilerParams(needs_layout_passes=False)

    @functools.partial(
        pl.kernel,
        mesh=mesh,
        compiler_params=cp,
        out_type=jax.ShapeDtypeStruct((B,), jnp.float32),
        scratch_types=[
            pltpu.VMEM((b_per_w // 128, 128), jnp.int32),
            pltpu.VMEM((b_per_w // 128, 128), jnp.int32),
            pltpu.VMEM((_R, nf1, 8, 128), jnp.float32),
            pltpu.VMEM((_R, nf1, 8, 128), jnp.float32),
            pltpu.VMEM((b_per_w,), jnp.float32),
            pltpu.VMEM((b_per_w,), jnp.float32),
            pltpu.VMEM((128,), jnp.int32),
            pltpu.VMEM((b_per_w,), jnp.float32),
            pltpu.SemaphoreType.DMA,
            pltpu.SemaphoreType.DMA,
            pltpu.SemaphoreType.DMA,
            pltpu.SemaphoreType.DMA,
            pltpu.SemaphoreType.DMA,
        ],
    )
    def k(u3r, i3r, ub_hbm, ib_hbm, u_idx_hbm, i_idx_hbm,
          out_hbm, uidx_v, iidx_v, u_ring, i_ring, ub_v, ib_v, drain_v,
          out_v, sem0, sem1, sem2, sem3, bsem):
        sems = [sem0, sem1, sem2, sem3]
        wid = lax.axis_index("s") * _NC + lax.axis_index("c")
        base = wid * b_per_w

        for kk in range(nrow):
            pltpu.async_copy(u_idx_hbm.at[pl.ds(base + kk * 128, 128)],
                             uidx_v.at[kk], bsem)
            pltpu.async_copy(i_idx_hbm.at[pl.ds(base + kk * 128, 128)],
                             iidx_v.at[kk], bsem)
        for kk in range(2 * nrow):
            pltpu.make_async_copy(u_idx_hbm.at[pl.ds(0, 128)],
                                  uidx_v.at[0], bsem).wait()

        for kk in range(nrow):
            pltpu.async_copy(ub_hbm.at[uidx_v.at[kk]],
                             ub_v.at[pl.ds(kk * 128, 128)], bsem)
            pltpu.async_copy(ib_hbm.at[iidx_v.at[kk]],
                             ib_v.at[pl.ds(kk * 128, 128)], bsem)

        def idx_vecs(g):
            rb = g * _G
            uidx16 = uidx_v[rb // 128, pl.ds(rb % 128, _G)]
            iidx16 = iidx_v[rb // 128, pl.ds(rb % 128, _G)]
            return uidx16, iidx16

        def enqueue(uidx16, iidx16, j, slot):
            uc = pl.multiple_of(
                lax.shift_right_logical(uidx16[j], 7) * jnp.int32(128), 128)
            ic = pl.multiple_of(
                lax.shift_right_logical(iidx16[j], 7) * jnp.int32(128), 128)
            pltpu.async_copy(u3r.at[:, :, pl.ds(uc, 128)],
                             u_ring.at[slot], sems[slot])
            pltpu.async_copy(i3r.at[:, :, pl.ds(ic, 128)],
                             i_ring.at[slot], sems[slot])

        def wait_slot(slot):
            pltpu.make_async_copy(u3r.at[:, :, pl.ds(0, 128)],
                                  u_ring.at[slot], sems[slot]).wait()
            pltpu.make_async_copy(i3r.at[:, :, pl.ds(0, 128)],
                                  i_ring.at[slot], sems[slot]).wait()

        lane = lax.iota(jnp.int32, _L)
        f2v = lax.bitwise_and(lane, 7)
        f1v = lax.shift_right_logical(lane, 3)

        def compute(uidx16, iidx16, j, slot, out16):
            ucb = jnp.zeros((_L,), jnp.int32) + lax.bitwise_and(uidx16[j], 127)
            icb = jnp.zeros((_L,), jnp.int32) + lax.bitwise_and(iidx16[j], 127)
            acc = None
            for q in range(F // _L):
                f1q = f1v + (2 * q)
                uv = plsc.load_gather(u_ring.at[slot], [f1q, f2v, ucb])
                iv = plsc.load_gather(i_ring.at[slot], [f1q, f2v, icb])
                prod = uv * iv
                acc = prod if acc is None else acc + prod
            return out16 + jnp.where(lane == j, jnp.sum(acc), 0.0)

        u0, i0 = idx_vecs(0)
        enqueue(u0, i0, 0, 0)
        enqueue(u0, i0, 1, 1)
        enqueue(u0, i0, 2, 2)

        for kk in range(2 * nrow):
            pltpu.make_async_copy(
                u_idx_hbm.at[pl.ds(0, 128)],
                drain_v.at[pl.ds(0, 128)], bsem).wait()

        @pl.loop(0, ng)
        def _(g):
            uidx16, iidx16 = idx_vecs(g)
            un, inn = idx_vecs(jnp.minimum(g + 1, ng - 1))
            rb = g * _G
            out16 = ub_v[pl.ds(rb, _G)] + ib_v[pl.ds(rb, _G)]
            for j in range(_G):
                slot = (j + 3) % _R
                if j < _G - 3:
                    enqueue(uidx16, iidx16, j + 3, slot)
                else:
                    @pl.when(g + 1 < ng)
                    def _():
                        enqueue(un, inn, j + 3 - _G, slot)
                wait_slot(j % _R)
                out16 = compute(uidx16, iidx16, j, j % _R, out16)
            out_v[pl.ds(rb, _G)] = out16

        pltpu.sync_copy(out_v, out_hbm.at[pl.ds(base, b_per_w)])

    return k(u3, i3, u_bias1, i_bias1, u_idx, i_idx)


@jax.jit
def kernel(u_emb, i_emb, u_bias, i_bias, u_idx, i_idx):
    B = u_idx.shape[0]
    F = u_emb.shape[1]
    N = u_emb.shape[0]
    u3 = u_emb.T.reshape(F // 8, 8, N)
    i3 = i_emb.T.reshape(F // 8, 8, N)
    ub1 = u_bias.reshape(-1)
    ib1 = i_bias.reshape(-1)
    return _mf_kernel(
        B, F, u3, i3, ub1, ib1,
        u_idx.astype(jnp.int32), i_idx.astype(jnp.int32),
    )
